# baseline (speedup 1.0000x reference)
import functools

import jax
import jax.numpy as jnp
from jax import lax
from jax.experimental import pallas as pl
from jax.experimental.pallas import tpu as pltpu
from jax.experimental.pallas import tpu_sc as plsc

N = 10000
E = 320000
D = 128
NC = 2
NS = 16
NW = NC * NS
E2 = 327680
CH = 64
RPT = (E2 // NW) // CH
SB = 20
EG = E2 // (SB * CH)
DST_PAD = 10100
NPAD = 10240
NPT = NPAD // NS
NACC = 10112
NRT = NACC // NS
RB = 2000
GRID = N // RB

_mesh = functools.partial(
    plsc.VectorSubcoreMesh,
    core_axis_name="c",
    subcore_axis_name="s",
    num_cores=NC,
    num_subcores=NS,
)


def _sc_degcnt_body(dst_hbm, w_hbm, deg_out, cnt_out, dstb, wb, oneb, zb,
                    acc_deg, acc_cnt, semd, semc):
    cid = lax.axis_index("c")
    sid = lax.axis_index("s")
    for i in range(CH // 16):
        oneb[pl.ds(i * 16, 16)] = jnp.full((16,), 1.0, jnp.float32)
    for i in range(NPT // 16):
        zb[pl.ds(i * 16, 16)] = jnp.zeros((16,), jnp.float32)
    pltpu.sync_copy(zb, acc_deg.at[pl.ds(sid * NPT, NPT)])
    pltpu.sync_copy(zb, acc_cnt.at[pl.ds(sid * NPT, NPT)])
    plsc.subcore_barrier()
    base = cid * (EG // NC) + sid * (RPT // SB)

    def superchunk(s, carry):
        g = base + s
        pltpu.sync_copy(dst_hbm.at[g], dstb)
        pltpu.sync_copy(w_hbm.at[g], wb)
        descs = []
        for j in range(SB):
            descs.append(pltpu.async_copy(
                wb.at[j], acc_deg.at[dstb.at[j]], semd, add=True))
            descs.append(pltpu.async_copy(
                oneb, acc_cnt.at[dstb.at[j]], semc, add=True))
        for d in descs:
            d.wait()
        return carry

    lax.fori_loop(0, RPT // SB, superchunk, 0)
    plsc.subcore_barrier()
    pltpu.sync_copy(acc_deg.at[pl.ds(sid * NPT, NPT)],
                    deg_out.at[cid, pl.ds(sid * NPT, NPT)])
    pltpu.sync_copy(acc_cnt.at[pl.ds(sid * NPT, NPT)],
                    cnt_out.at[cid, pl.ds(sid * NPT, NPT)])


_sc_degcnt = functools.partial(
    pl.kernel,
    out_type=(
        jax.ShapeDtypeStruct((NC, NPAD), jnp.float32),
        jax.ShapeDtypeStruct((NC, NPAD), jnp.float32),
    ),
    mesh=_mesh(),
    scratch_types=[
        pltpu.VMEM((SB, CH), jnp.int32),
        pltpu.VMEM((SB, CH), jnp.float32),
        pltpu.VMEM((CH,), jnp.float32),
        pltpu.VMEM((NPT,), jnp.float32),
        pltpu.VMEM_SHARED((NPAD,), jnp.float32),
        pltpu.VMEM_SHARED((NPAD,), jnp.float32),
        pltpu.SemaphoreType.DMA,
        pltpu.SemaphoreType.DMA,
    ],
)(_sc_degcnt_body)


NBUF = 4


def _edge_pass_body(scale, *refs):
    if scale:
        (src_hbm, dst_hbm, w_hbm, feat_hbm, zeros_hbm, out_hbm,
         srcb, dstb, wb, r0, r1, r2, r3, acc,
         gs0, gs1, gs2, gs3, ss0, ss1, ss2, ss3) = refs
    else:
        (src_hbm, dst_hbm, feat_hbm, zeros_hbm, out_hbm,
         srcb, dstb, r0, r1, r2, r3, acc,
         gs0, gs1, gs2, gs3, ss0, ss1, ss2, ss3) = refs
    cid = lax.axis_index("c")
    sid = lax.axis_index("s")
    pltpu.sync_copy(zeros_hbm.at[pl.ds(sid * NRT, NRT)],
                    acc.at[pl.ds(sid * NRT, NRT)])
    plsc.subcore_barrier()
    base = cid * (EG // NC) + sid * (RPT // SB)
    bufs = (r0, r1, r2, r3)
    gsems = (gs0, gs1, gs2, gs3)
    ssems = (ss0, ss1, ss2, ss3)

    def do_scale(rbuf, c):
        if not scale:
            return
        for g in range(CH // 16):
            grp = wb[c, pl.ds(g * 16, 16)]
            for r in range(16):
                k = g * 16 + r
                wkv = lax.gather(
                    grp, jnp.full((16, 1), r, jnp.int32),
                    lax.GatherDimensionNumbers(
                        offset_dims=(), collapsed_slice_dims=(0,),
                        start_index_map=(0,)),
                    (1,), mode=lax.GatherScatterMode.PROMISE_IN_BOUNDS)
                for j in range(D // 16):
                    rbuf[k, pl.ds(j * 16, 16)] = (
                        rbuf[k, pl.ds(j * 16, 16)] * wkv)

    def start_gather(c, b):
        pltpu.async_copy(feat_hbm.at[srcb.at[c]], bufs[b], gsems[b])

    def wait_gather(b):
        pltpu.make_async_copy(feat_hbm.at[srcb.at[0]], bufs[b],
                              gsems[b]).wait()

    def start_scatter(c, b):
        pltpu.async_copy(bufs[b], acc.at[dstb.at[c]], ssems[b], add=True)

    def wait_scatter(b):
        pltpu.make_async_copy(bufs[b], acc.at[dstb.at[0]], ssems[b]).wait()

    def superchunk(s, carry):
        g = base + s
        pltpu.sync_copy(src_hbm.at[g], srcb)
        pltpu.sync_copy(dst_hbm.at[g], dstb)
        if scale:
            pltpu.sync_copy(w_hbm.at[g], wb)
        start_gather(0, 0)
        start_gather(1, 1)
        wait_gather(0)
        start_gather(2, 2)
        do_scale(r0, 0)
        start_scatter(0, 0)
        wait_gather(1)
        start_gather(3, 3)
        do_scale(r1, 1)
        start_scatter(1, 1)

        def quad(k, c2):
            c0 = 4 * k + 2
            for u in range(4):
                c = c0 + u
                b = (2 + u) % NBUF
                bn = (b + 2) % NBUF
                wait_gather(b)
                wait_scatter(bn)
                start_gather(c + 2, bn)
                do_scale(bufs[b], c)
                start_scatter(c, b)
            return c2

        lax.fori_loop(0, (SB - 4) // 4, quad, 0)
        b2 = (SB - 2) % NBUF
        b3 = (SB - 1) % NBUF
        wait_gather(b2)
        wait_scatter((b2 + 2) % NBUF)
        do_scale(bufs[b2], SB - 2)
        start_scatter(SB - 2, b2)
        wait_gather(b3)
        wait_scatter((b3 + 2) % NBUF)
        do_scale(bufs[b3], SB - 1)
        start_scatter(SB - 1, b3)
        wait_scatter(b2)
        wait_scatter(b3)
        return carry

    lax.fori_loop(0, RPT // SB, superchunk, 0)
    plsc.subcore_barrier()
    pltpu.sync_copy(acc.at[pl.ds(sid * NRT, NRT)],
                    out_hbm.at[cid, pl.ds(sid * NRT, NRT)])


def _make_edge_pass(scale):
    scratch = [
        pltpu.VMEM((SB, CH), jnp.int32),
        pltpu.VMEM((SB, CH), jnp.int32),
    ]
    if scale:
        scratch.append(pltpu.VMEM((SB, CH), jnp.float32))
    scratch += (
        [pltpu.VMEM((CH, D), jnp.float32)] * 4
        + [pltpu.VMEM_SHARED((NACC, D), jnp.float32)]
        + [pltpu.SemaphoreType.DMA] * 8
    )
    return functools.partial(
        pl.kernel,
        out_type=jax.ShapeDtypeStruct((NC, NACC, D), jnp.float32),
        mesh=_mesh(),
        scratch_types=scratch,
    )(functools.partial(_edge_pass_body, scale))


_sc_pass1 = _make_edge_pass(True)
_sc_pass2 = _make_edge_pass(False)


def _tc1_body(x_ref, dp_ref, w_ref, b_ref, hp_ref):
    deg = dp_ref[0, 0, :] + dp_ref[0, 1, :]
    dinv = lax.rsqrt(jnp.maximum(deg, 1e-12))
    h = jnp.dot(x_ref[...], w_ref[...],
                preferred_element_type=jnp.float32) + b_ref[...]
    hp_ref[...] = h * dinv[:, None]


def _tc2_body(pp_ref, dp_ref, hg_ref):
    deg = dp_ref[0, 0, :] + dp_ref[0, 1, :]
    dinv = lax.rsqrt(jnp.maximum(deg, 1e-12))
    hg_ref[...] = (pp_ref[0] + pp_ref[1]) * dinv[:, None]


def _tc3_body(qq_ref, cp_ref, hg_ref, wl_ref, wr_ref, b_ref, out_ref):
    cnt = jnp.maximum(cp_ref[0, 0, :] + cp_ref[0, 1, :], 1.0)
    mean = (qq_ref[0] + qq_ref[1]) / cnt[:, None]
    out = (jnp.dot(mean, wl_ref[...], preferred_element_type=jnp.float32)
           + jnp.dot(hg_ref[...], wr_ref[...], preferred_element_type=jnp.float32)
           + b_ref[...])
    nrm = jnp.sqrt(jnp.sum(out * out, axis=-1, keepdims=True))
    out_ref[...] = out / jnp.maximum(nrm, 1e-12)


def kernel(x, edge_index, attr, W_gcn, b_gcn, W_l, W_r, b_sage):
    pad = E2 - E
    src = jnp.concatenate(
        [edge_index[0].astype(jnp.int32), jnp.zeros((pad,), jnp.int32)]
    ).reshape(EG, SB, CH)
    dst = jnp.concatenate(
        [edge_index[1].astype(jnp.int32),
         jnp.full((pad,), DST_PAD, jnp.int32)]
    ).reshape(EG, SB, CH)
    w = jnp.concatenate(
        [attr.reshape(-1).astype(jnp.float32), jnp.zeros((pad,), jnp.float32)]
    ).reshape(EG, SB, CH)
    zeros = jnp.zeros((NACC, D), jnp.float32)
    b_gcn2 = b_gcn.reshape(1, D)
    b_sage2 = b_sage.reshape(1, D)

    dp, cp = _sc_degcnt(dst, w)
    dps = dp[:, :N].reshape(NC, GRID, RB).transpose(1, 0, 2)
    cps = cp[:, :N].reshape(NC, GRID, RB).transpose(1, 0, 2)

    hp = pl.pallas_call(
        _tc1_body,
        grid=(GRID,),
        in_specs=[
            pl.BlockSpec((RB, D), lambda i: (i, 0)),
            pl.BlockSpec((1, NC, RB), lambda i: (i, 0, 0)),
            pl.BlockSpec((D, D), lambda i: (0, 0)),
            pl.BlockSpec((1, D), lambda i: (0, 0)),
        ],
        out_specs=pl.BlockSpec((RB, D), lambda i: (i, 0)),
        out_shape=jax.ShapeDtypeStruct((N, D), jnp.float32),
    )(x, dps, W_gcn, b_gcn2)

    pp = _sc_pass1(src, dst, w, hp, zeros)

    hg = pl.pallas_call(
        _tc2_body,
        grid=(GRID,),
        in_specs=[
            pl.BlockSpec((NC, RB, D), lambda i: (0, i, 0)),
            pl.BlockSpec((1, NC, RB), lambda i: (i, 0, 0)),
        ],
        out_specs=pl.BlockSpec((RB, D), lambda i: (i, 0)),
        out_shape=jax.ShapeDtypeStruct((N, D), jnp.float32),
    )(pp, dps)

    qq = _sc_pass2(src, dst, hg, zeros)

    out = pl.pallas_call(
        _tc3_body,
        grid=(GRID,),
        in_specs=[
            pl.BlockSpec((NC, RB, D), lambda i: (0, i, 0)),
            pl.BlockSpec((1, NC, RB), lambda i: (i, 0, 0)),
            pl.BlockSpec((RB, D), lambda i: (i, 0)),
            pl.BlockSpec((D, D), lambda i: (0, 0)),
            pl.BlockSpec((D, D), lambda i: (0, 0)),
            pl.BlockSpec((1, D), lambda i: (0, 0)),
        ],
        out_specs=pl.BlockSpec((RB, D), lambda i: (i, 0)),
        out_shape=jax.ShapeDtypeStruct((N, D), jnp.float32),
    )(qq, cps, hg, W_l, W_r, b_sage2)

    return out


# speedup vs baseline: 2.2388x; 2.2388x over previous
import functools

import jax
import jax.numpy as jnp
from jax import lax
from jax.experimental import pallas as pl
from jax.experimental.pallas import tpu as pltpu
from jax.experimental.pallas import tpu_sc as plsc

N = 10000
E = 320000
D = 128
NC = 2
NS = 16
NW = NC * NS
CH = 80
RPT = (E // NW) // CH
SB = 25
EG = E // (SB * CH)
NPAD = 10240
NPT = NPAD // NS
NACC = 10240
NRT = NACC // NS
RB = 2000
GRID = N // RB

_mesh = functools.partial(
    plsc.VectorSubcoreMesh,
    core_axis_name="c",
    subcore_axis_name="s",
    num_cores=NC,
    num_subcores=NS,
)


def _sc_degcnt_body(dst_hbm, w_hbm, deg_out, cnt_out, dstb, wb, oneb, zb,
                    acc_deg, acc_cnt, semd, semc):
    cid = lax.axis_index("c")
    sid = lax.axis_index("s")
    for i in range(CH // 16):
        oneb[pl.ds(i * 16, 16)] = jnp.full((16,), 1.0, jnp.float32)
    for i in range(NPT // 16):
        zb[pl.ds(i * 16, 16)] = jnp.zeros((16,), jnp.float32)
    pltpu.sync_copy(zb, acc_deg.at[pl.ds(sid * NPT, NPT)])
    pltpu.sync_copy(zb, acc_cnt.at[pl.ds(sid * NPT, NPT)])
    plsc.subcore_barrier()
    base = cid * (EG // NC) + sid * (RPT // SB)

    def superchunk(s, carry):
        g = base + s
        pltpu.sync_copy(dst_hbm.at[g], dstb)
        pltpu.sync_copy(w_hbm.at[g], wb)
        descs = []
        for j in range(SB):
            descs.append(pltpu.async_copy(
                wb.at[j], acc_deg.at[dstb.at[j]], semd, add=True))
            descs.append(pltpu.async_copy(
                oneb, acc_cnt.at[dstb.at[j]], semc, add=True))
        for d in descs:
            d.wait()
        return carry

    lax.fori_loop(0, RPT // SB, superchunk, 0)
    plsc.subcore_barrier()
    pltpu.sync_copy(acc_deg.at[pl.ds(sid * NPT, NPT)],
                    deg_out.at[cid, pl.ds(sid * NPT, NPT)])
    pltpu.sync_copy(acc_cnt.at[pl.ds(sid * NPT, NPT)],
                    cnt_out.at[cid, pl.ds(sid * NPT, NPT)])


_sc_degcnt = functools.partial(
    pl.kernel,
    out_type=(
        jax.ShapeDtypeStruct((NC, NPAD), jnp.float32),
        jax.ShapeDtypeStruct((NC, NPAD), jnp.float32),
    ),
    mesh=_mesh(),
    scratch_types=[
        pltpu.VMEM((SB, CH), jnp.int32),
        pltpu.VMEM((SB, CH), jnp.float32),
        pltpu.VMEM((CH,), jnp.float32),
        pltpu.VMEM((NPT,), jnp.float32),
        pltpu.VMEM_SHARED((NPAD,), jnp.float32),
        pltpu.VMEM_SHARED((NPAD,), jnp.float32),
        pltpu.SemaphoreType.DMA,
        pltpu.SemaphoreType.DMA,
    ],
)(_sc_degcnt_body)


def _edge_pass_body(scale, *refs):
    if scale:
        (src_hbm, dst_hbm, w_hbm, feat_hbm, zeros_hbm, out_hbm,
         srcb, dstb, wb, r0, r1, acc, gs0, gs1, ss0, ss1) = refs
    else:
        (src_hbm, dst_hbm, feat_hbm, zeros_hbm, out_hbm,
         srcb, dstb, r0, r1, acc, gs0, gs1, ss0, ss1) = refs
    cid = lax.axis_index("c")
    sid = lax.axis_index("s")
    pltpu.sync_copy(zeros_hbm.at[pl.ds(sid * NRT, NRT)],
                    acc.at[pl.ds(sid * NRT, NRT)])
    plsc.subcore_barrier()
    base = cid * (EG // NC) + sid * (RPT // SB)
    bufs = (r0, r1)
    gsems = (gs0, gs1)
    ssems = (ss0, ss1)

    def do_scale(rbuf, c):
        if not scale:
            return
        for g in range(CH // 16):
            grp = wb[c, pl.ds(g * 16, 16)]
            for r in range(16):
                k = g * 16 + r
                wkv = lax.gather(
                    grp, jnp.full((16, 1), r, jnp.int32),
                    lax.GatherDimensionNumbers(
                        offset_dims=(), collapsed_slice_dims=(0,),
                        start_index_map=(0,)),
                    (1,), mode=lax.GatherScatterMode.PROMISE_IN_BOUNDS)
                for j in range(D // 16):
                    rbuf[k, pl.ds(j * 16, 16)] = (
                        rbuf[k, pl.ds(j * 16, 16)] * wkv)

    def start_gather(c, b):
        pltpu.async_copy(feat_hbm.at[srcb.at[c]], bufs[b], gsems[b])

    def wait_gather(b):
        pltpu.make_async_copy(feat_hbm.at[srcb.at[0]], bufs[b],
                              gsems[b]).wait()

    def start_scatter(c, b):
        pltpu.async_copy(bufs[b], acc.at[dstb.at[c]], ssems[b], add=True)

    def wait_scatter(b):
        pltpu.make_async_copy(bufs[b], acc.at[dstb.at[0]], ssems[b]).wait()

    def superchunk(s, carry):
        g = base + s
        pltpu.sync_copy(src_hbm.at[g], srcb)
        pltpu.sync_copy(dst_hbm.at[g], dstb)
        if scale:
            pltpu.sync_copy(w_hbm.at[g], wb)
        start_gather(0, 0)
        wait_gather(0)
        start_gather(1, 1)
        do_scale(r0, 0)
        start_scatter(0, 0)

        def pair(p, c2):
            c = 2 * p + 1
            wait_gather(1)
            wait_scatter(0)
            start_gather(c + 1, 0)
            do_scale(r1, c)
            start_scatter(c, 1)
            wait_gather(0)
            wait_scatter(1)
            start_gather(c + 2, 1)
            do_scale(r0, c + 1)
            start_scatter(c + 1, 0)
            return c2

        lax.fori_loop(0, (SB - 3) // 2, pair, 0)
        wait_gather(1)
        wait_scatter(0)
        start_gather(SB - 1, 0)
        do_scale(r1, SB - 2)
        start_scatter(SB - 2, 1)
        wait_gather(0)
        wait_scatter(1)
        do_scale(r0, SB - 1)
        start_scatter(SB - 1, 0)
        wait_scatter(0)
        return carry

    lax.fori_loop(0, RPT // SB, superchunk, 0)
    plsc.subcore_barrier()
    pltpu.sync_copy(acc.at[pl.ds(sid * NRT, NRT)],
                    out_hbm.at[cid, pl.ds(sid * NRT, NRT)])


def _make_edge_pass(scale):
    scratch = [
        pltpu.VMEM((SB, CH), jnp.int32),
        pltpu.VMEM((SB, CH), jnp.int32),
    ]
    if scale:
        scratch.append(pltpu.VMEM((SB, CH), jnp.float32))
    scratch += (
        [pltpu.VMEM((CH, D), jnp.float32)] * 2
        + [pltpu.VMEM_SHARED((NACC, D), jnp.float32)]
        + [pltpu.SemaphoreType.DMA] * 4
    )
    return functools.partial(
        pl.kernel,
        out_type=jax.ShapeDtypeStruct((NC, NACC, D), jnp.float32),
        mesh=_mesh(),
        scratch_types=scratch,
    )(functools.partial(_edge_pass_body, scale))


_sc_pass1 = _make_edge_pass(True)
_sc_pass2 = _make_edge_pass(False)


def _tc1_body(x_ref, dp_ref, w_ref, b_ref, hp_ref):
    deg = dp_ref[0, 0, :] + dp_ref[0, 1, :]
    dinv = lax.rsqrt(jnp.maximum(deg, 1e-12))
    h = jnp.dot(x_ref[...], w_ref[...],
                preferred_element_type=jnp.float32) + b_ref[...]
    hp_ref[...] = h * dinv[:, None]


def _tc2_body(pp_ref, dp_ref, hg_ref):
    deg = dp_ref[0, 0, :] + dp_ref[0, 1, :]
    dinv = lax.rsqrt(jnp.maximum(deg, 1e-12))
    hg_ref[...] = (pp_ref[0] + pp_ref[1]) * dinv[:, None]


def _tc3_body(qq_ref, cp_ref, hg_ref, wl_ref, wr_ref, b_ref, out_ref):
    cnt = jnp.maximum(cp_ref[0, 0, :] + cp_ref[0, 1, :], 1.0)
    mean = (qq_ref[0] + qq_ref[1]) / cnt[:, None]
    out = (jnp.dot(mean, wl_ref[...], preferred_element_type=jnp.float32)
           + jnp.dot(hg_ref[...], wr_ref[...], preferred_element_type=jnp.float32)
           + b_ref[...])
    nrm = jnp.sqrt(jnp.sum(out * out, axis=-1, keepdims=True))
    out_ref[...] = out / jnp.maximum(nrm, 1e-12)


def kernel(x, edge_index, attr, W_gcn, b_gcn, W_l, W_r, b_sage):
    src = edge_index[0].astype(jnp.int32).reshape(EG, SB, CH)
    dst = edge_index[1].astype(jnp.int32).reshape(EG, SB, CH)
    w = attr.reshape(-1).astype(jnp.float32).reshape(EG, SB, CH)
    zeros = jnp.zeros((NACC, D), jnp.float32)
    b_gcn2 = b_gcn.reshape(1, D)
    b_sage2 = b_sage.reshape(1, D)

    dp, cp = _sc_degcnt(dst, w)
    dps = dp[:, :N].reshape(NC, GRID, RB).transpose(1, 0, 2)
    cps = cp[:, :N].reshape(NC, GRID, RB).transpose(1, 0, 2)

    hp = pl.pallas_call(
        _tc1_body,
        grid=(GRID,),
        in_specs=[
            pl.BlockSpec((RB, D), lambda i: (i, 0)),
            pl.BlockSpec((1, NC, RB), lambda i: (i, 0, 0)),
            pl.BlockSpec((D, D), lambda i: (0, 0)),
            pl.BlockSpec((1, D), lambda i: (0, 0)),
        ],
        out_specs=pl.BlockSpec((RB, D), lambda i: (i, 0)),
        out_shape=jax.ShapeDtypeStruct((N, D), jnp.float32),
    )(x, dps, W_gcn, b_gcn2)

    pp = _sc_pass1(src, dst, w, hp, zeros)

    hg = pl.pallas_call(
        _tc2_body,
        grid=(GRID,),
        in_specs=[
            pl.BlockSpec((NC, RB, D), lambda i: (0, i, 0)),
            pl.BlockSpec((1, NC, RB), lambda i: (i, 0, 0)),
        ],
        out_specs=pl.BlockSpec((RB, D), lambda i: (i, 0)),
        out_shape=jax.ShapeDtypeStruct((N, D), jnp.float32),
    )(pp, dps)

    qq = _sc_pass2(src, dst, hg, zeros)

    out = pl.pallas_call(
        _tc3_body,
        grid=(GRID,),
        in_specs=[
            pl.BlockSpec((NC, RB, D), lambda i: (0, i, 0)),
            pl.BlockSpec((1, NC, RB), lambda i: (i, 0, 0)),
            pl.BlockSpec((RB, D), lambda i: (i, 0)),
            pl.BlockSpec((D, D), lambda i: (0, 0)),
            pl.BlockSpec((D, D), lambda i: (0, 0)),
            pl.BlockSpec((1, D), lambda i: (0, 0)),
        ],
        out_specs=pl.BlockSpec((RB, D), lambda i: (i, 0)),
        out_shape=jax.ShapeDtypeStruct((N, D), jnp.float32),
    )(qq, cps, hg, W_l, W_r, b_sage2)

    return out
